# Initial kernel scaffold; baseline (speedup 1.0000x reference)
import jax
import jax.numpy as jnp
from jax.experimental import pallas as pl


def kernel(edge_attr, W0, W1, W2):
    raise NotImplementedError("write your pallas kernel here")


# speedup vs baseline: 1.9484x; 1.9484x over previous
import jax
import jax.numpy as jnp
from jax import lax
from jax.experimental import pallas as pl
from jax.experimental.pallas import tpu as pltpu
from jax.experimental.pallas import tpu_sc as plsc

DIM = 128
NC, NS, L = 2, 16, 16
NW = NC * NS
E = 320000
PER_W = E // NW
GR = 80
KG = 5
CH = GR * KG
NCHUNK = PER_W // CH
S0, S1 = 21, 3


def _body(a0_hbm, a1_hbm, a2_hbm, tab_hbm, out_hbm,
          a0_v, a1_v, a2_v, c0, c1, c2, c3, c4, rows_v, sem):
    wid = lax.axis_index("s") * NC + lax.axis_index("c")
    base = wid * PER_W
    cbufs = [c0, c1, c2, c3, c4]

    def chunk(g, carry):
        off = pl.multiple_of(base + g * CH, CH)
        pltpu.sync_copy(a0_hbm.at[pl.ds(off, CH)], a0_v)
        pltpu.sync_copy(a1_hbm.at[pl.ds(off, CH)], a1_v)
        pltpu.sync_copy(a2_hbm.at[pl.ds(off, CH)], a2_v)
        for k in range(KG):
            for j in range(GR // L):
                s = pl.ds(k * GR + j * L, L)
                cbufs[k][pl.ds(j * L, L)] = a0_v[s] * S0 + a1_v[s] * S1 + a2_v[s]
        copies = [
            pltpu.async_copy(tab_hbm.at[cbufs[k]],
                             rows_v.at[pl.ds(k * GR, GR)], sem)
            for k in range(KG)
        ]
        for c in copies:
            c.wait()
        pltpu.sync_copy(rows_v, out_hbm.at[pl.ds(off, CH)])
        return carry

    lax.fori_loop(0, NCHUNK, chunk, 0)


def kernel(edge_attr, W0, W1, W2):
    ea = edge_attr.astype(jnp.int32)
    a0, a1, a2 = ea[:, 0], ea[:, 1], ea[:, 2]
    tab = (W0[:, None, None, :] + W1[None, :, None, :]
           + W2[None, None, :, :]).reshape(-1, DIM)
    tab = jnp.pad(tab, ((0, 128 - tab.shape[0]), (0, 0)))

    mesh = plsc.VectorSubcoreMesh(core_axis_name="c", subcore_axis_name="s")
    f = pl.kernel(
        _body,
        out_type=jax.ShapeDtypeStruct((E, DIM), jnp.float32),
        mesh=mesh,
        scratch_types=[
            pltpu.VMEM((CH,), jnp.int32),
            pltpu.VMEM((CH,), jnp.int32),
            pltpu.VMEM((CH,), jnp.int32),
            pltpu.VMEM((GR,), jnp.int32),
            pltpu.VMEM((GR,), jnp.int32),
            pltpu.VMEM((GR,), jnp.int32),
            pltpu.VMEM((GR,), jnp.int32),
            pltpu.VMEM((GR,), jnp.int32),
            pltpu.VMEM((CH, DIM), jnp.float32),
            pltpu.SemaphoreType.DMA,
        ],
    )
    return f(a0, a1, a2, tab)


# speedup vs baseline: 1.9502x; 1.0010x over previous
import jax
import jax.numpy as jnp
from jax import lax
from jax.experimental import pallas as pl
from jax.experimental.pallas import tpu as pltpu
from jax.experimental.pallas import tpu_sc as plsc

DIM = 128
NC, NS, L = 2, 16, 16
NW = NC * NS
E = 320000
PER_W = E // NW
GR = 80
KG = 5
CH = GR * KG
NCHUNK = PER_W // CH
S0, S1 = 21, 3


def _body(a0_hbm, a1_hbm, a2_hbm, tab_hbm, out_hbm,
          a0v0, a1v0, a2v0, a0v1, a1v1, a2v1,
          c00, c01, c02, c03, c04, c10, c11, c12, c13, c14,
          rows0, rows1, isem, gsem0, gsem1, ssem0, ssem1):
    wid = lax.axis_index("s") * NC + lax.axis_index("c")
    base = pl.multiple_of(wid * PER_W, CH)
    av = [(a0v0, a1v0, a2v0), (a0v1, a1v1, a2v1)]
    cb = [[c00, c01, c02, c03, c04], [c10, c11, c12, c13, c14]]
    rows = [rows0, rows1]
    gsem = [gsem0, gsem1]
    ssem = [ssem0, ssem1]

    def off_of(g):
        return pl.multiple_of(base + g * CH, 8)

    def load_and_prep(g, b):
        a0v, a1v, a2v = av[b]
        off = off_of(g)
        d0 = pltpu.async_copy(a0_hbm.at[pl.ds(off, CH)], a0v, isem)
        d1 = pltpu.async_copy(a1_hbm.at[pl.ds(off, CH)], a1v, isem)
        d2 = pltpu.async_copy(a2_hbm.at[pl.ds(off, CH)], a2v, isem)
        d0.wait(); d1.wait(); d2.wait()
        for k in range(KG):
            for j in range(GR // L):
                s = pl.ds(k * GR + j * L, L)
                cb[b][k][pl.ds(j * L, L)] = a0v[s] * S0 + a1v[s] * S1 + a2v[s]

    def fire_gathers(b):
        for k in range(KG):
            pltpu.async_copy(tab_hbm.at[cb[b][k]],
                             rows[b].at[pl.ds(k * GR, GR)], gsem[b])

    def drain_gathers(b):
        for k in range(KG):
            pltpu.make_async_copy(tab_hbm.at[cb[b][k]],
                                  rows[b].at[pl.ds(k * GR, GR)], gsem[b]).wait()

    def fire_store(g, b):
        pltpu.async_copy(rows[b], out_hbm.at[pl.ds(off_of(g), CH)], ssem[b])

    def drain_store(b):
        pltpu.make_async_copy(rows[b], out_hbm.at[pl.ds(base, CH)], ssem[b]).wait()

    load_and_prep(0, 0)
    fire_gathers(0)
    load_and_prep(1, 1)
    fire_gathers(1)
    drain_gathers(0)
    fire_store(0, 0)

    def pair(i, carry):
        ga = 2 + 2 * i
        load_and_prep(ga, 0)
        drain_store(0)
        fire_gathers(0)
        drain_gathers(1)
        fire_store(ga - 1, 1)
        load_and_prep(ga + 1, 1)
        drain_store(1)
        fire_gathers(1)
        drain_gathers(0)
        fire_store(ga, 0)
        return carry

    lax.fori_loop(0, (NCHUNK - 3) // 2, pair, 0)

    gl = NCHUNK - 1
    load_and_prep(gl, 0)
    drain_store(0)
    fire_gathers(0)
    drain_gathers(1)
    fire_store(gl - 1, 1)
    drain_gathers(0)
    fire_store(gl, 0)
    drain_store(1)
    drain_store(0)


def kernel(edge_attr, W0, W1, W2):
    ea = edge_attr.astype(jnp.int32)
    a0, a1, a2 = ea[:, 0], ea[:, 1], ea[:, 2]
    tab = (W0[:, None, None, :] + W1[None, :, None, :]
           + W2[None, None, :, :]).reshape(-1, DIM)
    tab = jnp.pad(tab, ((0, 128 - tab.shape[0]), (0, 0)))

    mesh = plsc.VectorSubcoreMesh(core_axis_name="c", subcore_axis_name="s")
    f = pl.kernel(
        _body,
        out_type=jax.ShapeDtypeStruct((E, DIM), jnp.float32),
        mesh=mesh,
        scratch_types=(
            [pltpu.VMEM((CH,), jnp.int32) for _ in range(6)]
            + [pltpu.VMEM((GR,), jnp.int32) for _ in range(10)]
            + [pltpu.VMEM((CH, DIM), jnp.float32) for _ in range(2)]
            + [pltpu.SemaphoreType.DMA for _ in range(5)]
        ),
    )
    return f(a0, a1, a2, tab)


# speedup vs baseline: 19.2658x; 9.8787x over previous
import jax
import jax.numpy as jnp
from jax import lax
from jax.experimental import pallas as pl
from jax.experimental.pallas import tpu as pltpu
from jax.experimental.pallas import tpu_sc as plsc

DIM = 128
NC, NS, L = 2, 16, 16
NW = NC * NS
E = 320000
PER_W = E // NW
GR = 80
KG = 5
CH = GR * KG
NCHUNK = PER_W // CH
S0, S1 = 21, 3


def _body(a0_hbm, a1_hbm, a2_hbm, tab_hbm, out_hbm,
          a0v0, a1v0, a2v0, a0v1, a1v1, a2v1,
          c00, c01, c02, c03, c04, c10, c11, c12, c13, c14,
          rows0, rows1, tab_v, isem, gsem0, gsem1, ssem0, ssem1):
    wid = lax.axis_index("s") * NC + lax.axis_index("c")
    base = pl.multiple_of(wid * PER_W, CH)
    @pl.when(lax.axis_index("s") == 0)
    def _stage():
        pltpu.sync_copy(tab_hbm, tab_v)
    plsc.subcore_barrier()
    av = [(a0v0, a1v0, a2v0), (a0v1, a1v1, a2v1)]
    cb = [[c00, c01, c02, c03, c04], [c10, c11, c12, c13, c14]]
    rows = [rows0, rows1]
    gsem = [gsem0, gsem1]
    ssem = [ssem0, ssem1]

    def off_of(g):
        return pl.multiple_of(base + g * CH, 8)

    def load_and_prep(g, b):
        a0v, a1v, a2v = av[b]
        off = off_of(g)
        d0 = pltpu.async_copy(a0_hbm.at[pl.ds(off, CH)], a0v, isem)
        d1 = pltpu.async_copy(a1_hbm.at[pl.ds(off, CH)], a1v, isem)
        d2 = pltpu.async_copy(a2_hbm.at[pl.ds(off, CH)], a2v, isem)
        d0.wait(); d1.wait(); d2.wait()
        for k in range(KG):
            for j in range(GR // L):
                s = pl.ds(k * GR + j * L, L)
                cb[b][k][pl.ds(j * L, L)] = a0v[s] * S0 + a1v[s] * S1 + a2v[s]

    def fire_gathers(b):
        for k in range(KG):
            pltpu.async_copy(tab_v.at[cb[b][k]],
                             rows[b].at[pl.ds(k * GR, GR)], gsem[b])

    def drain_gathers(b):
        for k in range(KG):
            pltpu.make_async_copy(tab_v.at[cb[b][k]],
                                  rows[b].at[pl.ds(k * GR, GR)], gsem[b]).wait()

    def fire_store(g, b):
        pltpu.async_copy(rows[b], out_hbm.at[pl.ds(off_of(g), CH)], ssem[b])

    def drain_store(b):
        pltpu.make_async_copy(rows[b], out_hbm.at[pl.ds(base, CH)], ssem[b]).wait()

    load_and_prep(0, 0)
    fire_gathers(0)
    load_and_prep(1, 1)
    fire_gathers(1)
    drain_gathers(0)
    fire_store(0, 0)

    def pair(i, carry):
        ga = 2 + 2 * i
        load_and_prep(ga, 0)
        drain_store(0)
        fire_gathers(0)
        drain_gathers(1)
        fire_store(ga - 1, 1)
        load_and_prep(ga + 1, 1)
        drain_store(1)
        fire_gathers(1)
        drain_gathers(0)
        fire_store(ga, 0)
        return carry

    lax.fori_loop(0, (NCHUNK - 3) // 2, pair, 0)

    gl = NCHUNK - 1
    load_and_prep(gl, 0)
    drain_store(0)
    fire_gathers(0)
    drain_gathers(1)
    fire_store(gl - 1, 1)
    drain_gathers(0)
    fire_store(gl, 0)
    drain_store(1)
    drain_store(0)


def kernel(edge_attr, W0, W1, W2):
    ea = edge_attr.astype(jnp.int32)
    a0, a1, a2 = ea[:, 0], ea[:, 1], ea[:, 2]
    tab = (W0[:, None, None, :] + W1[None, :, None, :]
           + W2[None, None, :, :]).reshape(-1, DIM)
    tab = jnp.pad(tab, ((0, 128 - tab.shape[0]), (0, 0)))

    mesh = plsc.VectorSubcoreMesh(core_axis_name="c", subcore_axis_name="s")
    f = pl.kernel(
        _body,
        out_type=jax.ShapeDtypeStruct((E, DIM), jnp.float32),
        mesh=mesh,
        scratch_types=(
            [pltpu.VMEM((CH,), jnp.int32) for _ in range(6)]
            + [pltpu.VMEM((GR,), jnp.int32) for _ in range(10)]
            + [pltpu.VMEM((CH, DIM), jnp.float32) for _ in range(2)]
            + [pltpu.VMEM_SHARED((128, DIM), jnp.float32)]
            + [pltpu.SemaphoreType.DMA for _ in range(5)]
        ),
    )
    return f(a0, a1, a2, tab)


# speedup vs baseline: 19.8030x; 1.0279x over previous
import jax
import jax.numpy as jnp
from jax import lax
from jax.experimental import pallas as pl
from jax.experimental.pallas import tpu as pltpu
from jax.experimental.pallas import tpu_sc as plsc

DIM = 128
NC, NS, L = 2, 16, 16
NW = NC * NS
E = 320000
PER_W = E // NW
GR = 80
NG = PER_W // GR
NB = 5
S0, S1 = 21, 3


def _body(a0_hbm, a1_hbm, a2_hbm, tab_hbm, out_hbm,
          a0v, a1v, a2v, cidx, r0, r1, r2, r3, r4, tab_sp,
          isem, g0, g1, g2, g3, g4, s0, s1, s2, s3, s4):
    wid = lax.axis_index("s") * NC + lax.axis_index("c")
    base = pl.multiple_of(wid * PER_W, GR)
    rows = [r0, r1, r2, r3, r4]
    gsem = [g0, g1, g2, g3, g4]
    ssem = [s0, s1, s2, s3, s4]

    @pl.when(lax.axis_index("s") == 0)
    def _stage():
        pltpu.sync_copy(tab_hbm, tab_sp)

    d0 = pltpu.async_copy(a0_hbm.at[pl.ds(base, PER_W)], a0v, isem)
    d1 = pltpu.async_copy(a1_hbm.at[pl.ds(base, PER_W)], a1v, isem)
    d2 = pltpu.async_copy(a2_hbm.at[pl.ds(base, PER_W)], a2v, isem)
    d0.wait(); d1.wait(); d2.wait()
    plsc.subcore_barrier()

    def cidx_slice(t):
        return cidx.at[pl.ds(pl.multiple_of(t * GR, 8), GR)]

    def prep(t, b):
        for j in range(GR // L):
            s = pl.ds(pl.multiple_of(t * GR + j * L, 8), L)
            cidx[s] = a0v[s] * S0 + a1v[s] * S1 + a2v[s]

    def fire_gather(t, b):
        pltpu.async_copy(tab_sp.at[cidx_slice(t)], rows[b], gsem[b])

    def finish(t, b):
        pltpu.make_async_copy(tab_sp.at[cidx_slice(t)], rows[b], gsem[b]).wait()
        off = pl.multiple_of(base + t * GR, 8)
        pltpu.async_copy(rows[b], out_hbm.at[pl.ds(off, GR)], ssem[b])

    def drain_store(b):
        pltpu.make_async_copy(rows[b], out_hbm.at[pl.ds(base, GR)], ssem[b]).wait()

    for t in range(NB):
        prep(t, t)
        fire_gather(t, t)
        if t >= 1:
            finish(t - 1, t - 1)

    def outer(i, carry):
        t0 = NB + i * NB
        for u in range(NB):
            t = t0 + u
            b = u
            prep(t, b)
            drain_store(b)
            fire_gather(t, b)
            bp = (u - 1) % NB
            finish(t - 1, bp)
        return carry

    lax.fori_loop(0, NG // NB - 1, outer, 0)

    finish(NG - 1, NB - 1)
    for b in range(NB):
        drain_store(b)


def kernel(edge_attr, W0, W1, W2):
    ea = edge_attr.astype(jnp.int32)
    a0, a1, a2 = ea[:, 0], ea[:, 1], ea[:, 2]
    tab = (W0[:, None, None, :] + W1[None, :, None, :]
           + W2[None, None, :, :]).reshape(-1, DIM)
    tab = jnp.pad(tab, ((0, 128 - tab.shape[0]), (0, 0)))

    mesh = plsc.VectorSubcoreMesh(core_axis_name="c", subcore_axis_name="s")
    f = pl.kernel(
        _body,
        out_type=jax.ShapeDtypeStruct((E, DIM), jnp.float32),
        mesh=mesh,
        scratch_types=(
            [pltpu.VMEM((PER_W,), jnp.int32) for _ in range(4)]
            + [pltpu.VMEM((GR, DIM), jnp.float32) for _ in range(NB)]
            + [pltpu.VMEM_SHARED((128, DIM), jnp.float32)]
            + [pltpu.SemaphoreType.DMA for _ in range(11)]
        ),
    )
    return f(a0, a1, a2, tab)


# speedup vs baseline: 19.9521x; 1.0075x over previous
import jax
import jax.numpy as jnp
from jax import lax
from jax.experimental import pallas as pl
from jax.experimental.pallas import tpu as pltpu
from jax.experimental.pallas import tpu_sc as plsc

DIM = 128
NC, NS, L = 2, 16, 16
NW = NC * NS
E = 320000
PER_W = E // NW
GR = 80
NG = PER_W // GR
NB = 5
LAG = 2
S0, S1 = 21, 3


def _body(a0_hbm, a1_hbm, a2_hbm, tab_hbm, out_hbm,
          a0v, a1v, a2v, cidx, r0, r1, r2, r3, r4, tab_sp,
          isem, g0, g1, g2, g3, g4, s0, s1, s2, s3, s4):
    wid = lax.axis_index("s") * NC + lax.axis_index("c")
    base = pl.multiple_of(wid * PER_W, GR)
    rows = [r0, r1, r2, r3, r4]
    gsem = [g0, g1, g2, g3, g4]
    ssem = [s0, s1, s2, s3, s4]

    @pl.when(lax.axis_index("s") == 0)
    def _stage():
        pltpu.sync_copy(tab_hbm, tab_sp)

    d0 = pltpu.async_copy(a0_hbm.at[pl.ds(base, PER_W)], a0v, isem)
    d1 = pltpu.async_copy(a1_hbm.at[pl.ds(base, PER_W)], a1v, isem)
    d2 = pltpu.async_copy(a2_hbm.at[pl.ds(base, PER_W)], a2v, isem)
    d0.wait(); d1.wait(); d2.wait()
    plsc.subcore_barrier()

    def cidx_slice(t):
        return cidx.at[pl.ds(pl.multiple_of(t * GR, 8), GR)]

    def prep(t, b):
        for j in range(GR // L):
            s = pl.ds(pl.multiple_of(t * GR + j * L, 8), L)
            cidx[s] = a0v[s] * S0 + a1v[s] * S1 + a2v[s]

    def fire_gather(t, b):
        pltpu.async_copy(tab_sp.at[cidx_slice(t)], rows[b], gsem[b])

    def finish(t, b):
        pltpu.make_async_copy(tab_sp.at[cidx_slice(t)], rows[b], gsem[b]).wait()
        off = pl.multiple_of(base + t * GR, 8)
        pltpu.async_copy(rows[b], out_hbm.at[pl.ds(off, GR)], ssem[b])

    def drain_store(b):
        pltpu.make_async_copy(rows[b], out_hbm.at[pl.ds(base, GR)], ssem[b]).wait()

    for t in range(NB):
        prep(t, t)
        fire_gather(t, t)
        if t >= LAG:
            finish(t - LAG, t - LAG)

    def outer(i, carry):
        t0 = NB + i * NB
        for u in range(NB):
            t = t0 + u
            b = u
            prep(t, b)
            drain_store(b)
            fire_gather(t, b)
            finish(t - LAG, (u - LAG) % NB)
        return carry

    lax.fori_loop(0, NG // NB - 1, outer, 0)

    for t in range(NG - LAG, NG):
        finish(t, t % NB)
    for b in range(NB):
        drain_store(b)


def kernel(edge_attr, W0, W1, W2):
    ea = edge_attr.astype(jnp.int32)
    a0, a1, a2 = ea[:, 0], ea[:, 1], ea[:, 2]
    tab = (W0[:, None, None, :] + W1[None, :, None, :]
           + W2[None, None, :, :]).reshape(-1, DIM)
    tab = jnp.pad(tab, ((0, 128 - tab.shape[0]), (0, 0)))

    mesh = plsc.VectorSubcoreMesh(core_axis_name="c", subcore_axis_name="s")
    f = pl.kernel(
        _body,
        out_type=jax.ShapeDtypeStruct((E, DIM), jnp.float32),
        mesh=mesh,
        scratch_types=(
            [pltpu.VMEM((PER_W,), jnp.int32) for _ in range(4)]
            + [pltpu.VMEM((GR, DIM), jnp.float32) for _ in range(NB)]
            + [pltpu.VMEM_SHARED((128, DIM), jnp.float32)]
            + [pltpu.SemaphoreType.DMA for _ in range(11)]
        ),
    )
    return f(a0, a1, a2, tab)


# speedup vs baseline: 20.0177x; 1.0033x over previous
import jax
import jax.numpy as jnp
from jax import lax
from jax.experimental import pallas as pl
from jax.experimental.pallas import tpu as pltpu
from jax.experimental.pallas import tpu_sc as plsc

DIM = 128
NC, NS, L = 2, 16, 16
NW = NC * NS
E = 320000
PER_W = E // NW
GR = 80
NG = PER_W // GR
NB = 5
LAG = 2
S0, S1 = 21, 3


def _body(a0_hbm, a1_hbm, a2_hbm, tab_hbm, out_hbm,
          a0v, a1v, a2v, cidx, r0, r1, r2, r3, r4, tab_sp,
          isem, g0, g1, g2, g3, g4, s0, s1, s2, s3, s4):
    wid = lax.axis_index("s") * NC + lax.axis_index("c")
    base = pl.multiple_of(wid * PER_W, GR)
    rows = [r0, r1, r2, r3, r4]
    gsem = [g0, g1, g2, g3, g4]
    ssem = [s0, s1, s2, s3, s4]

    @pl.when(lax.axis_index("s") == 0)
    def _stage():
        pltpu.sync_copy(tab_hbm, tab_sp)

    d0 = pltpu.async_copy(a0_hbm.at[pl.ds(base, PER_W)], a0v, isem)
    d1 = pltpu.async_copy(a1_hbm.at[pl.ds(base, PER_W)], a1v, isem)
    d2 = pltpu.async_copy(a2_hbm.at[pl.ds(base, PER_W)], a2v, isem)
    d0.wait(); d1.wait(); d2.wait()
    plsc.subcore_barrier()

    def cidx_slice(t):
        return cidx.at[pl.ds(pl.multiple_of(t * GR, 8), GR)]

    def prep(t, b):
        for j in range(GR // L):
            s = pl.ds(pl.multiple_of(t * GR + j * L, 8), L)
            cidx[s] = a0v[s] * S0 + a1v[s] * S1 + a2v[s]

    def fire_gather(t, b):
        pltpu.async_copy(tab_sp.at[cidx_slice(t)], rows[b], gsem[b])

    def finish(t, b):
        pltpu.make_async_copy(tab_sp.at[cidx_slice(t)], rows[b], gsem[b]).wait()
        off = pl.multiple_of(base + t * GR, 8)
        pltpu.async_copy(rows[b], out_hbm.at[pl.ds(off, GR)], ssem[b])

    def drain_store(b):
        pltpu.make_async_copy(rows[b], out_hbm.at[pl.ds(base, GR)], ssem[b]).wait()

    for t in range(NB):
        prep(t, t)
        fire_gather(t, t)
        if t >= LAG:
            finish(t - LAG, t - LAG)

    def outer(i, carry):
        t0 = NB + i * NB
        for u in range(NB):
            t = t0 + u
            b = u
            prep(t, b)
            finish(t - LAG, (u - LAG) % NB)
            drain_store(b)
            fire_gather(t, b)
        return carry

    lax.fori_loop(0, NG // NB - 1, outer, 0)

    for t in range(NG - LAG, NG):
        finish(t, t % NB)
    for b in range(NB):
        drain_store(b)


def kernel(edge_attr, W0, W1, W2):
    ea = edge_attr.astype(jnp.int32)
    a0, a1, a2 = ea[:, 0], ea[:, 1], ea[:, 2]
    tab = (W0[:, None, None, :] + W1[None, :, None, :]
           + W2[None, None, :, :]).reshape(-1, DIM)
    tab = jnp.pad(tab, ((0, 128 - tab.shape[0]), (0, 0)))

    mesh = plsc.VectorSubcoreMesh(core_axis_name="c", subcore_axis_name="s")
    f = pl.kernel(
        _body,
        out_type=jax.ShapeDtypeStruct((E, DIM), jnp.float32),
        mesh=mesh,
        scratch_types=(
            [pltpu.VMEM((PER_W,), jnp.int32) for _ in range(4)]
            + [pltpu.VMEM((GR, DIM), jnp.float32) for _ in range(NB)]
            + [pltpu.VMEM_SHARED((128, DIM), jnp.float32)]
            + [pltpu.SemaphoreType.DMA for _ in range(11)]
        ),
    )
    return f(a0, a1, a2, tab)


# speedup vs baseline: 20.3003x; 1.0141x over previous
import jax
import jax.numpy as jnp
from jax import lax
from jax.experimental import pallas as pl
from jax.experimental.pallas import tpu as pltpu
from jax.experimental.pallas import tpu_sc as plsc

DIM = 128
NC, NS, L = 2, 16, 16
NW = NC * NS
E = 320000
PER_W = E // NW
GR = 80
NG = PER_W // GR
NB = 5
LAG = 2
HEAD = 2 * NB * GR
S0, S1 = 21, 3


def _body(a0_hbm, a1_hbm, a2_hbm, tab_hbm, out_hbm,
          a0v, a1v, a2v, cidx, rb0, rb1, rb2, rb3, rb4, tab_sp,
          isem, rsem, g0, g1, g2, g3, g4, s0, s1, s2, s3, s4):
    wid = lax.axis_index("s") * NC + lax.axis_index("c")
    base = pl.multiple_of(wid * PER_W, GR)
    rows = [rb0, rb1, rb2, rb3, rb4]
    gsem = [g0, g1, g2, g3, g4]
    ssem = [s0, s1, s2, s3, s4]

    d0 = pltpu.async_copy(a0_hbm.at[pl.ds(base, HEAD)], a0v.at[pl.ds(0, HEAD)], isem)
    d1 = pltpu.async_copy(a1_hbm.at[pl.ds(base, HEAD)], a1v.at[pl.ds(0, HEAD)], isem)
    d2 = pltpu.async_copy(a2_hbm.at[pl.ds(base, HEAD)], a2v.at[pl.ds(0, HEAD)], isem)
    base_r = pl.multiple_of(base + HEAD, 8)
    r0 = pltpu.async_copy(a0_hbm.at[pl.ds(base_r, PER_W - HEAD)],
                          a0v.at[pl.ds(HEAD, PER_W - HEAD)], rsem)
    r1 = pltpu.async_copy(a1_hbm.at[pl.ds(base_r, PER_W - HEAD)],
                          a1v.at[pl.ds(HEAD, PER_W - HEAD)], rsem)
    r2 = pltpu.async_copy(a2_hbm.at[pl.ds(base_r, PER_W - HEAD)],
                          a2v.at[pl.ds(HEAD, PER_W - HEAD)], rsem)
    sid = lax.axis_index("s")
    srow = pl.multiple_of(sid * (128 // NS), 8)
    pltpu.sync_copy(tab_hbm.at[pl.ds(srow, 128 // NS)],
                    tab_sp.at[pl.ds(srow, 128 // NS)])
    d0.wait(); d1.wait(); d2.wait()
    plsc.subcore_barrier()

    def cidx_slice(t):
        return cidx.at[pl.ds(pl.multiple_of(t * GR, 8), GR)]

    def prep(t, b):
        for j in range(GR // L):
            s = pl.ds(pl.multiple_of(t * GR + j * L, 8), L)
            cidx[s] = a0v[s] * S0 + a1v[s] * S1 + a2v[s]

    def fire_gather(t, b):
        pltpu.async_copy(tab_sp.at[cidx_slice(t)], rows[b], gsem[b])

    def finish(t, b):
        pltpu.make_async_copy(tab_sp.at[cidx_slice(t)], rows[b], gsem[b]).wait()
        off = pl.multiple_of(base + t * GR, 8)
        pltpu.async_copy(rows[b], out_hbm.at[pl.ds(off, GR)], ssem[b])

    def drain_store(b):
        pltpu.make_async_copy(rows[b], out_hbm.at[pl.ds(base, GR)], ssem[b]).wait()

    for t in range(NB):
        prep(t, t)
        fire_gather(t, t)
        if t >= LAG:
            finish(t - LAG, t - LAG)
    r0.wait(); r1.wait(); r2.wait()

    def outer(i, carry):
        t0 = NB + i * NB
        for u in range(NB):
            t = t0 + u
            b = u
            prep(t, b)
            finish(t - LAG, (u - LAG) % NB)
            drain_store(b)
            fire_gather(t, b)
        return carry

    lax.fori_loop(0, NG // NB - 1, outer, 0)

    for t in range(NG - LAG, NG):
        finish(t, t % NB)
    for b in range(NB):
        drain_store(b)


def kernel(edge_attr, W0, W1, W2):
    ea = edge_attr.astype(jnp.int32)
    a0, a1, a2 = ea[:, 0], ea[:, 1], ea[:, 2]
    tab = (W0[:, None, None, :] + W1[None, :, None, :]
           + W2[None, None, :, :]).reshape(-1, DIM)
    tab = jnp.pad(tab, ((0, 128 - tab.shape[0]), (0, 0)))

    mesh = plsc.VectorSubcoreMesh(core_axis_name="c", subcore_axis_name="s")
    f = pl.kernel(
        _body,
        out_type=jax.ShapeDtypeStruct((E, DIM), jnp.float32),
        mesh=mesh,
        scratch_types=(
            [pltpu.VMEM((PER_W,), jnp.int32) for _ in range(4)]
            + [pltpu.VMEM((GR, DIM), jnp.float32) for _ in range(NB)]
            + [pltpu.VMEM_SHARED((128, DIM), jnp.float32)]
            + [pltpu.SemaphoreType.DMA for _ in range(12)]
        ),
    )
    return f(a0, a1, a2, tab)
